# baseline (speedup 1.0000x reference)
import jax
import jax.numpy as jnp
from jax.experimental import pallas as pl
from jax.experimental.pallas import tpu as pltpu

_DELAY = 8388608
_FEEDBACK = 0.5
_COLS = 1024
_ROWS = _DELAY // _COLS
_BLOCK_ROWS = 512


def _body(x_ref, idx_ref, buf_ref, out_s_ref, out_buf_ref):
    j = pl.program_id(0)
    blk = buf_ref[...]
    idx = idx_ref[0]
    row = idx // _COLS
    col = idx - row * _COLS
    lrow = row - j * _BLOCK_ROWS
    own = (lrow >= 0) & (lrow < _BLOCK_ROWS)
    out_buf_ref[...] = blk

    @pl.when(own)
    def _update():
        x = x_ref[0]
        ri = jax.lax.broadcasted_iota(jnp.int32, (_BLOCK_ROWS, _COLS), 0)
        ci = jax.lax.broadcasted_iota(jnp.int32, (_BLOCK_ROWS, _COLS), 1)
        mask = (ri == lrow) & (ci == col)
        bo = jnp.sum(jnp.where(mask, blk, 0.0))
        out_s_ref[0] = -x + bo
        out_buf_ref[...] = jnp.where(mask, x + bo * _FEEDBACK, blk)


def kernel(x, buffer, buffer_index):
    buf2 = buffer.reshape(_ROWS, _COLS)
    idx = jnp.asarray(buffer_index, jnp.int32).reshape(1)
    xs = x.reshape(1).astype(jnp.float32)
    out_s, out_buf = pl.pallas_call(
        _body,
        grid=(_ROWS // _BLOCK_ROWS,),
        in_specs=[
            pl.BlockSpec(memory_space=pltpu.SMEM),
            pl.BlockSpec(memory_space=pltpu.SMEM),
            pl.BlockSpec((_BLOCK_ROWS, _COLS), lambda j: (j, 0)),
        ],
        out_specs=[
            pl.BlockSpec(memory_space=pltpu.SMEM),
            pl.BlockSpec((_BLOCK_ROWS, _COLS), lambda j: (j, 0)),
        ],
        out_shape=[
            jax.ShapeDtypeStruct((1,), jnp.float32),
            jax.ShapeDtypeStruct((_ROWS, _COLS), jnp.float32),
        ],
    )(xs, idx, buf2)
    return (out_s[0], out_buf.reshape(_DELAY))
